# baseline (speedup 1.0000x reference)
import functools
import math

import jax
import jax.numpy as jnp
from jax import lax
from jax.experimental import pallas as pl
from jax.experimental.pallas import tpu as pltpu
from jax.experimental.pallas import tpu_sc as plsc

D_MODEL = 1024
MAX_LEN = 8192
BATCH = 4
SEQ = 2048
SCALE = math.sqrt(D_MODEL)

NC = 2
NS = 16
NW = NC * NS
ROWS = BATCH * SEQ
ROWS_PER_W = ROWS // NW
CH = 16
NCHUNK = ROWS_PER_W // CH
NSLOT = 4
G = NCHUNK // NSLOT
LANES = 16


def _sc_body(idx_hbm, pe_hbm, pos_hbm, idx_v, pe_b, sem_in, sem_out):
    wid = lax.axis_index("c") * NS + lax.axis_index("s")
    row0 = wid * ROWS_PER_W

    pltpu.sync_copy(idx_hbm.at[pl.ds(row0, ROWS_PER_W)], idx_v)

    def start_in(i, k):
        pltpu.async_copy(pe_hbm.at[idx_v.at[pl.ds(i * CH, CH)]],
                         pe_b[k], sem_in[k])

    def wait_in(i, k):
        pltpu.make_async_copy(pe_hbm.at[idx_v.at[pl.ds(i * CH, CH)]],
                              pe_b[k], sem_in[k]).wait()

    def start_pos(i, k):
        pltpu.async_copy(pe_b[k], pos_hbm.at[pl.ds(row0 + i * CH, CH), :],
                         sem_out[k])

    def drain_pos(i, k):
        pltpu.make_async_copy(pe_b[k], pos_hbm.at[pl.ds(row0 + i * CH, CH), :],
                              sem_out[k]).wait()

    for k in range(NSLOT):
        start_in(k, k)

    def outer(g, carry):
        for k in range(NSLOT):
            i = g * NSLOT + k
            if k == 0:
                @pl.when(g > 0)
                def _reuse_prev():
                    drain_pos(i - 1, NSLOT - 1)
                    start_in(i - 1 + NSLOT, NSLOT - 1)
            else:
                drain_pos(i - 1, k - 1)

                @pl.when(i - 1 + NSLOT < NCHUNK)
                def _refill():
                    start_in(i - 1 + NSLOT, k - 1)

            wait_in(i, k)
            start_pos(i, k)
        return carry

    lax.fori_loop(0, G, outer, 0)
    drain_pos(NCHUNK - 1, NSLOT - 1)


def _sc_wrap(idx_hbm, pe_hbm, pos_hbm, idx_v, *rest):
    _body_groups = [rest[j * NSLOT:(j + 1) * NSLOT] for j in range(3)]
    _sc_body(idx_hbm, pe_hbm, pos_hbm, idx_v, *_body_groups)


def _sc_pos(idx, pe):
    mesh = plsc.VectorSubcoreMesh(core_axis_name="c", subcore_axis_name="s")
    return pl.kernel(
        _sc_wrap,
        out_type=jax.ShapeDtypeStruct((ROWS, D_MODEL), jnp.float32),
        mesh=mesh,
        scratch_types=(
            [pltpu.VMEM((ROWS_PER_W,), jnp.int32)]
            + [pltpu.VMEM((CH, D_MODEL), jnp.float32)] * NSLOT
            + [pltpu.SemaphoreType.DMA] * (2 * NSLOT)
        ),
    )(idx, pe)


TC_ROWS = 256
TC_STEPS = ROWS // TC_ROWS
TC_PER_B = SEQ // TC_ROWS
TC_E = TC_ROWS * D_MODEL


def _tc_body(off_smem, x_ref, pe_any, out_ref, pe_buf, sem):
    i = pl.program_id(0)

    def start(j):
        b = j // TC_PER_B
        src = (off_smem[b] + (j % TC_PER_B) * TC_ROWS) * D_MODEL
        pltpu.make_async_copy(
            pe_any.at[pl.ds(src, TC_E)], pe_buf.at[j % 2], sem.at[j % 2]
        ).start()

    def wait(j):
        b = j // TC_PER_B
        src = (off_smem[b] + (j % TC_PER_B) * TC_ROWS) * D_MODEL
        pltpu.make_async_copy(
            pe_any.at[pl.ds(src, TC_E)], pe_buf.at[j % 2], sem.at[j % 2]
        ).wait()

    @pl.when(i == 0)
    def _prime():
        start(0)

    @pl.when(i + 1 < TC_STEPS)
    def _prefetch():
        start(i + 1)

    wait(i)
    out_ref[...] = x_ref[...] * SCALE + pe_buf[i % 2]


def _tc_out(offset, xf, pef):
    grid_spec = pltpu.PrefetchScalarGridSpec(
        num_scalar_prefetch=1,
        grid=(TC_STEPS,),
        in_specs=[
            pl.BlockSpec((TC_E,), lambda i, off: (i,)),
            pl.BlockSpec(memory_space=pl.ANY),
        ],
        out_specs=pl.BlockSpec((TC_E,), lambda i, off: (i,)),
        scratch_shapes=[
            pltpu.VMEM((2, TC_E), jnp.float32),
            pltpu.SemaphoreType.DMA((2,)),
        ],
    )
    return pl.pallas_call(
        _tc_body,
        grid_spec=grid_spec,
        out_shape=jax.ShapeDtypeStruct((ROWS * D_MODEL,), jnp.float32),
    )(offset, xf, pef)


@jax.jit
def _run(x2d, idx, offset, pe):
    pos = _sc_pos(idx, pe)
    out = _tc_out(offset, x2d.reshape(-1), pe.reshape(-1))
    return out, pos


def kernel(x, offset, pe):
    assert x.shape == (BATCH, SEQ, D_MODEL)
    x2d = x.reshape(ROWS, D_MODEL)
    offset = jnp.maximum(offset.astype(jnp.int32), 0)
    index = offset[:, None] + jnp.arange(SEQ, dtype=jnp.int32)
    index = index.reshape(ROWS)
    out_f, pos_f = _run(x2d, index, offset, pe)
    return out_f.reshape(x.shape), pos_f.reshape(x.shape)


# speedup vs baseline: 2.1707x; 2.1707x over previous
import functools
import math

import jax
import jax.numpy as jnp
from jax import lax
from jax.experimental import pallas as pl
from jax.experimental.pallas import tpu as pltpu
from jax.experimental.pallas import tpu_sc as plsc

D_MODEL = 1024
MAX_LEN = 8192
BATCH = 4
SEQ = 2048
SCALE = math.sqrt(D_MODEL)

NC = 2
NS = 16
NW = NC * NS
ROWS = BATCH * SEQ
ROWS_PER_W = ROWS // NW
CH = 16
NCHUNK = ROWS_PER_W // CH
NSLOT = 4
G = NCHUNK // NSLOT
LANES = 16


def _sc_body(idx_hbm, pe_hbm, pos_hbm, idx_v, pe_b, sem_in, sem_out):
    wid = lax.axis_index("c") * NS + lax.axis_index("s")
    row0 = wid * ROWS_PER_W

    pltpu.sync_copy(idx_hbm.at[pl.ds(row0, ROWS_PER_W)], idx_v)

    def start_in(i, k):
        pltpu.async_copy(pe_hbm.at[idx_v.at[pl.ds(i * CH, CH)]],
                         pe_b[k], sem_in[k])

    def wait_in(i, k):
        pltpu.make_async_copy(pe_hbm.at[idx_v.at[pl.ds(i * CH, CH)]],
                              pe_b[k], sem_in[k]).wait()

    def start_pos(i, k):
        pltpu.async_copy(pe_b[k], pos_hbm.at[pl.ds(row0 + i * CH, CH), :],
                         sem_out[k])

    def drain_pos(i, k):
        pltpu.make_async_copy(pe_b[k], pos_hbm.at[pl.ds(row0 + i * CH, CH), :],
                              sem_out[k]).wait()

    for k in range(NSLOT):
        start_in(k, k)

    def outer(g, carry):
        for k in range(NSLOT):
            i = g * NSLOT + k
            if k == 0:
                @pl.when(g > 0)
                def _reuse_prev():
                    drain_pos(i - 1, NSLOT - 1)
                    start_in(i - 1 + NSLOT, NSLOT - 1)
            else:
                drain_pos(i - 1, k - 1)

                @pl.when(i - 1 + NSLOT < NCHUNK)
                def _refill():
                    start_in(i - 1 + NSLOT, k - 1)

            wait_in(i, k)
            start_pos(i, k)
        return carry

    lax.fori_loop(0, G, outer, 0)
    drain_pos(NCHUNK - 1, NSLOT - 1)


def _sc_wrap(idx_hbm, pe_hbm, pos_hbm, idx_v, *rest):
    _body_groups = [rest[j * NSLOT:(j + 1) * NSLOT] for j in range(3)]
    _sc_body(idx_hbm, pe_hbm, pos_hbm, idx_v, *_body_groups)


def _sc_pos(idx, pe):
    mesh = plsc.VectorSubcoreMesh(core_axis_name="c", subcore_axis_name="s")
    return pl.kernel(
        _sc_wrap,
        out_type=jax.ShapeDtypeStruct((ROWS, D_MODEL), jnp.float32),
        mesh=mesh,
        scratch_types=(
            [pltpu.VMEM((ROWS_PER_W,), jnp.int32)]
            + [pltpu.VMEM((CH, D_MODEL), jnp.float32)] * NSLOT
            + [pltpu.SemaphoreType.DMA] * (2 * NSLOT)
        ),
    )(idx, pe)


TC_ROWS = 256
TC_STEPS = ROWS // TC_ROWS
TC_PER_B = SEQ // TC_ROWS
TC_PAD = 8


def _tc_body(off_smem, x_ref, pe_any, out_ref, pe_buf, sem):
    i = pl.program_id(0)

    def src_row(j):
        b = j // TC_PER_B
        off8 = (off_smem[b] // TC_PAD) * TC_PAD
        return off8 + (j % TC_PER_B) * TC_ROWS

    def start(j):
        pltpu.make_async_copy(
            pe_any.at[pl.ds(src_row(j), TC_ROWS + TC_PAD), :],
            pe_buf.at[j % 2], sem.at[j % 2],
        ).start()

    def wait(j):
        pltpu.make_async_copy(
            pe_any.at[pl.ds(src_row(j), TC_ROWS + TC_PAD), :],
            pe_buf.at[j % 2], sem.at[j % 2],
        ).wait()

    @pl.when(i == 0)
    def _prime():
        start(0)

    @pl.when(i + 1 < TC_STEPS)
    def _prefetch():
        start(i + 1)

    wait(i)
    b = i // TC_PER_B
    sh = off_smem[b] % TC_PAD
    for k in range(TC_PAD):
        @pl.when(sh == k)
        def _shifted(_k=k):
            out_ref[...] = (x_ref[...] * SCALE
                            + pe_buf[i % 2, _k:_k + TC_ROWS, :])


def _tc_out(offset, x2d, pe):
    grid_spec = pltpu.PrefetchScalarGridSpec(
        num_scalar_prefetch=1,
        grid=(TC_STEPS,),
        in_specs=[
            pl.BlockSpec((TC_ROWS, D_MODEL), lambda i, off: (i, 0)),
            pl.BlockSpec(memory_space=pl.ANY),
        ],
        out_specs=pl.BlockSpec((TC_ROWS, D_MODEL), lambda i, off: (i, 0)),
        scratch_shapes=[
            pltpu.VMEM((2, TC_ROWS + TC_PAD, D_MODEL), jnp.float32),
            pltpu.SemaphoreType.DMA((2,)),
        ],
    )
    return pl.pallas_call(
        _tc_body,
        grid_spec=grid_spec,
        out_shape=jax.ShapeDtypeStruct((ROWS, D_MODEL), jnp.float32),
    )(offset, x2d, pe)


@jax.jit
def _run(x2d, idx, offset, pe):
    pos = _sc_pos(idx, pe)
    out = _tc_out(offset, x2d, pe)
    return out, pos


def kernel(x, offset, pe):
    assert x.shape == (BATCH, SEQ, D_MODEL)
    x2d = x.reshape(ROWS, D_MODEL)
    offset = jnp.maximum(offset.astype(jnp.int32), 0)
    index = offset[:, None] + jnp.arange(SEQ, dtype=jnp.int32)
    index = index.reshape(ROWS)
    out_f, pos_f = _run(x2d, index, offset, pe)
    return out_f.reshape(x.shape), pos_f.reshape(x.shape)


# speedup vs baseline: 2.2967x; 1.0580x over previous
import functools
import math

import jax
import jax.numpy as jnp
from jax import lax
from jax.experimental import pallas as pl
from jax.experimental.pallas import tpu as pltpu
from jax.experimental.pallas import tpu_sc as plsc

D_MODEL = 1024
MAX_LEN = 8192
BATCH = 4
SEQ = 2048
SCALE = math.sqrt(D_MODEL)

NC = 2
NS = 16
NW = NC * NS
ROWS = BATCH * SEQ
ROWS_PER_W = ROWS // NW
CH = 16
NCHUNK = ROWS_PER_W // CH
NSLOT = 4
G = NCHUNK // NSLOT
LANES = 16


def _sc_body(idx_hbm, pe_hbm, pos_hbm, idx_v, pe_b, sem_in, sem_out):
    wid = lax.axis_index("c") * NS + lax.axis_index("s")
    row0 = wid * ROWS_PER_W

    pltpu.sync_copy(idx_hbm.at[pl.ds(row0, ROWS_PER_W)], idx_v)

    def start_in(i, k):
        pltpu.async_copy(pe_hbm.at[idx_v.at[pl.ds(i * CH, CH)]],
                         pe_b[k], sem_in[k])

    def wait_in(i, k):
        pltpu.make_async_copy(pe_hbm.at[idx_v.at[pl.ds(i * CH, CH)]],
                              pe_b[k], sem_in[k]).wait()

    def start_pos(i, k):
        pltpu.async_copy(pe_b[k], pos_hbm.at[pl.ds(row0 + i * CH, CH), :],
                         sem_out[k])

    def drain_pos(i, k):
        pltpu.make_async_copy(pe_b[k], pos_hbm.at[pl.ds(row0 + i * CH, CH), :],
                              sem_out[k]).wait()

    for k in range(NSLOT):
        start_in(k, k)

    def outer(g, carry):
        for k in range(NSLOT):
            i = g * NSLOT + k
            if k == 0:
                @pl.when(g > 0)
                def _reuse_prev():
                    drain_pos(i - 1, NSLOT - 1)
                    start_in(i - 1 + NSLOT, NSLOT - 1)
            else:
                drain_pos(i - 1, k - 1)

                @pl.when(i - 1 + NSLOT < NCHUNK)
                def _refill():
                    start_in(i - 1 + NSLOT, k - 1)

            wait_in(i, k)
            start_pos(i, k)
        return carry

    lax.fori_loop(0, G, outer, 0)
    drain_pos(NCHUNK - 1, NSLOT - 1)


def _sc_wrap(idx_hbm, pe_hbm, pos_hbm, idx_v, *rest):
    _body_groups = [rest[j * NSLOT:(j + 1) * NSLOT] for j in range(3)]
    _sc_body(idx_hbm, pe_hbm, pos_hbm, idx_v, *_body_groups)


def _sc_pos(idx, pe):
    mesh = plsc.VectorSubcoreMesh(core_axis_name="c", subcore_axis_name="s")
    return pl.kernel(
        _sc_wrap,
        out_type=jax.ShapeDtypeStruct((ROWS, D_MODEL), jnp.float32),
        mesh=mesh,
        scratch_types=(
            [pltpu.VMEM((ROWS_PER_W,), jnp.int32)]
            + [pltpu.VMEM((CH, D_MODEL), jnp.float32)] * NSLOT
            + [pltpu.SemaphoreType.DMA] * (2 * NSLOT)
        ),
    )(idx, pe)


TC_ROWS = 512
TC_STEPS = ROWS // TC_ROWS
TC_PER_B = SEQ // TC_ROWS
TC_PAD = 8


def _tc_body(off_smem, x_ref, pe_any, out_ref, pe_buf, sem):
    i = pl.program_id(0)

    def src_row(j):
        b = j // TC_PER_B
        off8 = (off_smem[b] // TC_PAD) * TC_PAD
        return off8 + (j % TC_PER_B) * TC_ROWS

    def start(j):
        pltpu.make_async_copy(
            pe_any.at[pl.ds(src_row(j), TC_ROWS + TC_PAD), :],
            pe_buf.at[j % 2], sem.at[j % 2],
        ).start()

    def wait(j):
        pltpu.make_async_copy(
            pe_any.at[pl.ds(src_row(j), TC_ROWS + TC_PAD), :],
            pe_buf.at[j % 2], sem.at[j % 2],
        ).wait()

    @pl.when(i == 0)
    def _prime():
        start(0)

    @pl.when(i + 1 < TC_STEPS)
    def _prefetch():
        start(i + 1)

    wait(i)
    b = i // TC_PER_B
    sh = off_smem[b] % TC_PAD
    for k in range(TC_PAD):
        @pl.when(sh == k)
        def _shifted(_k=k):
            out_ref[...] = (x_ref[...] * SCALE
                            + pe_buf[i % 2, _k:_k + TC_ROWS, :])


def _tc_out(offset, x2d, pe):
    grid_spec = pltpu.PrefetchScalarGridSpec(
        num_scalar_prefetch=1,
        grid=(TC_STEPS,),
        in_specs=[
            pl.BlockSpec((TC_ROWS, D_MODEL), lambda i, off: (i, 0)),
            pl.BlockSpec(memory_space=pl.ANY),
        ],
        out_specs=pl.BlockSpec((TC_ROWS, D_MODEL), lambda i, off: (i, 0)),
        scratch_shapes=[
            pltpu.VMEM((2, TC_ROWS + TC_PAD, D_MODEL), jnp.float32),
            pltpu.SemaphoreType.DMA((2,)),
        ],
    )
    return pl.pallas_call(
        _tc_body,
        grid_spec=grid_spec,
        out_shape=jax.ShapeDtypeStruct((ROWS, D_MODEL), jnp.float32),
    )(offset, x2d, pe)


@jax.jit
def _run(x2d, idx, offset, pe):
    pos = _sc_pos(idx, pe)
    out = _tc_out(offset, x2d, pe)
    return out, pos


def kernel(x, offset, pe):
    assert x.shape == (BATCH, SEQ, D_MODEL)
    x2d = x.reshape(ROWS, D_MODEL)
    offset = jnp.maximum(offset.astype(jnp.int32), 0)
    index = offset[:, None] + jnp.arange(SEQ, dtype=jnp.int32)
    index = index.reshape(ROWS)
    out_f, pos_f = _run(x2d, index, offset, pe)
    return out_f.reshape(x.shape), pos_f.reshape(x.shape)


# speedup vs baseline: 2.4239x; 1.0554x over previous
import math

import jax
import jax.numpy as jnp
from jax import lax
from jax.experimental import pallas as pl
from jax.experimental.pallas import tpu as pltpu
from jax.experimental.pallas import tpu_sc as plsc

D_MODEL = 1024
MAX_LEN = 8192
BATCH = 4
SEQ = 2048
SCALE = math.sqrt(D_MODEL)

NC = 2
NS = 16
NW = NC * NS
ROWS = BATCH * SEQ
ROWS_PER_W = ROWS // NW
CH = 8
NCHUNK = ROWS_PER_W // CH
NSLOT = 4
G = NCHUNK // NSLOT
U = 8
LANES = 16


def _body(x_hbm, idx_hbm, pe_hbm, out_hbm, pos_hbm,
          idx_v, pe_b, x_b, o_b, sem_in, sem_out):
    wid = lax.axis_index("c") * NS + lax.axis_index("s")
    row0 = wid * ROWS_PER_W

    pltpu.sync_copy(idx_hbm.at[pl.ds(row0, ROWS_PER_W)], idx_v)

    def start_in(i, k):
        pltpu.async_copy(pe_hbm.at[idx_v.at[pl.ds(i * CH, CH)]],
                         pe_b[k], sem_in[k])
        pltpu.async_copy(x_hbm.at[pl.ds(row0 + i * CH, CH), :],
                         x_b[k], sem_in[k])

    def wait_in(i, k):
        pltpu.make_async_copy(pe_hbm.at[idx_v.at[pl.ds(i * CH, CH)]],
                              pe_b[k], sem_in[k]).wait()
        pltpu.make_async_copy(x_hbm.at[pl.ds(row0 + i * CH, CH), :],
                              x_b[k], sem_in[k]).wait()

    for k in range(NSLOT):
        start_in(k, k)

    def outer(g, carry):
        for k in range(NSLOT):
            i = g * NSLOT + k
            wait_in(i, k)
            pltpu.async_copy(pe_b[k], pos_hbm.at[pl.ds(row0 + i * CH, CH), :],
                             sem_out[k])

            @pl.when(g > 0)
            def _drain_out():
                pltpu.make_async_copy(
                    o_b[k], out_hbm.at[pl.ds(row0 + i * CH, CH), :],
                    sem_out[k]).wait()

            for r in range(CH):
                @plsc.parallel_loop(0, D_MODEL // LANES, step=1, unroll=U)
                def _grp(c, _r=r, _k=k):
                    s = pl.ds(c * LANES, LANES)
                    o_b[_k][_r, s] = x_b[_k][_r, s] * SCALE + pe_b[_k][_r, s]

            pltpu.async_copy(o_b[k], out_hbm.at[pl.ds(row0 + i * CH, CH), :],
                             sem_out[k])

            pltpu.make_async_copy(
                pe_b[k], pos_hbm.at[pl.ds(row0 + i * CH, CH), :],
                sem_out[k]).wait()

            @pl.when(g < G - 1)
            def _prefetch():
                start_in(i + NSLOT, k)
        return carry

    lax.fori_loop(0, G, outer, 0)

    for k in range(NSLOT):
        i = (G - 1) * NSLOT + k
        pltpu.make_async_copy(o_b[k], out_hbm.at[pl.ds(row0 + i * CH, CH), :],
                              sem_out[k]).wait()


def _body_wrap(x_hbm, idx_hbm, pe_hbm, out_hbm, pos_hbm, idx_v, *rest):
    groups = [rest[j * NSLOT:(j + 1) * NSLOT] for j in range(5)]
    _body(x_hbm, idx_hbm, pe_hbm, out_hbm, pos_hbm, idx_v, *groups)


@jax.jit
def _sc_call(x2d, idx, pe):
    mesh = plsc.VectorSubcoreMesh(core_axis_name="c", subcore_axis_name="s")
    buf = pltpu.VMEM((CH, D_MODEL), jnp.float32)
    return pl.kernel(
        _body_wrap,
        out_type=(
            jax.ShapeDtypeStruct((ROWS, D_MODEL), jnp.float32),
            jax.ShapeDtypeStruct((ROWS, D_MODEL), jnp.float32),
        ),
        mesh=mesh,
        scratch_types=(
            [pltpu.VMEM((ROWS_PER_W,), jnp.int32)]
            + [buf] * (3 * NSLOT)
            + [pltpu.SemaphoreType.DMA] * (2 * NSLOT)
        ),
    )(x2d, idx, pe)


def kernel(x, offset, pe):
    assert x.shape == (BATCH, SEQ, D_MODEL)
    x2d = x.reshape(ROWS, D_MODEL)
    index = offset[:, None].astype(jnp.int32) + jnp.arange(SEQ, dtype=jnp.int32)
    index = jnp.maximum(index, 0).reshape(ROWS)
    out_f, pos_f = _sc_call(x2d, index, pe)
    return out_f.reshape(x.shape), pos_f.reshape(x.shape)
